# baseline (speedup 1.0000x reference)
import functools

import jax
import jax.numpy as jnp
from jax import lax
from jax.experimental import pallas as pl
from jax.experimental.pallas import tpu as pltpu
from jax.experimental.pallas import tpu_sc as plsc

BATCH = 16384
MF_DIM = 64
MLP_IN_HALF = 128
H0 = 128
H1 = 64

NC, NS = 2, 16
NW = NC * NS
B_PER_W = BATCH // NW
CHUNK = B_PER_W // 2


def _gather_all(gmf_user, gmf_item, mlp_user, mlp_item, user_idxs, item_idxs):
    mesh = plsc.VectorSubcoreMesh(core_axis_name="c", subcore_axis_name="s")

    @functools.partial(
        pl.kernel,
        mesh=mesh,
        out_type=[
            jax.ShapeDtypeStruct((BATCH, MF_DIM), jnp.float32),
            jax.ShapeDtypeStruct((BATCH, MF_DIM), jnp.float32),
            jax.ShapeDtypeStruct((BATCH, MLP_IN_HALF), jnp.float32),
            jax.ShapeDtypeStruct((BATCH, MLP_IN_HALF), jnp.float32),
        ],
        scratch_types=[
            pltpu.VMEM((B_PER_W,), jnp.int32),
            pltpu.VMEM((B_PER_W,), jnp.int32),
            pltpu.VMEM((CHUNK, MLP_IN_HALF), jnp.float32),
            pltpu.VMEM((CHUNK, MLP_IN_HALF), jnp.float32),
            pltpu.SemaphoreType.DMA,
            pltpu.SemaphoreType.DMA,
            pltpu.SemaphoreType.DMA,
        ],
    )
    def k(gu_hbm, gi_hbm, mu_hbm, mi_hbm, uidx_hbm, iidx_hbm,
          out_gu, out_gi, out_mu, out_mi,
          uidx_v, iidx_v, buf_a, buf_b, sg, s0, s1):
        wid = lax.axis_index("s") * NC + lax.axis_index("c")
        base = wid * B_PER_W
        sl = pl.ds(base, B_PER_W)
        pltpu.sync_copy(uidx_hbm.at[sl], uidx_v)
        pltpu.sync_copy(iidx_hbm.at[sl], iidx_v)
        @pl.loop(0, B_PER_W, step=16)
        def _(j):
            uvec = uidx_v[pl.ds(j, 16)]
            ivec = iidx_v[pl.ds(j, 16)]
            for t in range(16):
                pltpu.async_copy(gu_hbm.at[uvec[t]], out_gu.at[base + j + t], sg)
                pltpu.async_copy(gi_hbm.at[ivec[t]], out_gi.at[base + j + t], sg)

        c0 = pltpu.async_copy(mu_hbm.at[uidx_v.at[pl.ds(0, CHUNK)]], buf_a, s0)
        c1 = pltpu.async_copy(mu_hbm.at[uidx_v.at[pl.ds(CHUNK, CHUNK)]], buf_b, s1)
        c0.wait()
        pltpu.sync_copy(buf_a, out_mu.at[pl.ds(base, CHUNK)])
        c2 = pltpu.async_copy(mi_hbm.at[iidx_v.at[pl.ds(0, CHUNK)]], buf_a, s0)
        c1.wait()
        pltpu.sync_copy(buf_b, out_mu.at[pl.ds(base + CHUNK, CHUNK)])
        c3 = pltpu.async_copy(mi_hbm.at[iidx_v.at[pl.ds(CHUNK, CHUNK)]], buf_b, s1)
        c2.wait()
        pltpu.sync_copy(buf_a, out_mi.at[pl.ds(base, CHUNK)])
        c3.wait()
        pltpu.sync_copy(buf_b, out_mi.at[pl.ds(base + CHUNK, CHUNK)])

        pltpu.make_async_copy(gu_hbm.at[pl.ds(0, B_PER_W)], out_gu.at[sl], sg).wait()
        pltpu.make_async_copy(gi_hbm.at[pl.ds(0, B_PER_W)], out_gi.at[sl], sg).wait()

    return k(gmf_user, gmf_item, mlp_user, mlp_item, user_idxs, item_idxs)


BT = 2048


def _head_body(gu_ref, gi_ref, mu_ref, mi_ref,
               w0u_ref, w0i_ref, b0_ref, w1_ref, b1_ref,
               wfg_ref, wfm_ref, bf_ref, out_ref):
    h0 = jnp.dot(mu_ref[...], w0u_ref[...], preferred_element_type=jnp.float32)
    h0 += jnp.dot(mi_ref[...], w0i_ref[...], preferred_element_type=jnp.float32)
    h0 = jnp.maximum(h0 + b0_ref[...], 0.0)
    h1 = jnp.dot(h0, w1_ref[...], preferred_element_type=jnp.float32)
    h1 = jnp.maximum(h1 + b1_ref[...], 0.0)
    gmf = gu_ref[...] * gi_ref[...]
    logit = jnp.sum(gmf * wfg_ref[...], axis=1) + jnp.sum(h1 * wfm_ref[...], axis=1)
    out_ref[...] = jax.nn.sigmoid(logit + bf_ref[0])


def _dense_head(gu, gi, mu, mi, W0, b0, W1, b1, Wf, bf):
    w0u = W0[:, :MLP_IN_HALF].T
    w0i = W0[:, MLP_IN_HALF:].T
    w1 = W1.T
    wfg = Wf[0, :MF_DIM].reshape(1, MF_DIM)
    wfm = Wf[0, MF_DIM:].reshape(1, H1)
    b0r = b0.reshape(1, H0)
    b1r = b1.reshape(1, H1)

    grid = (BATCH // BT,)
    full = lambda shape: pl.BlockSpec(shape, lambda i: (0,) * len(shape))
    return pl.pallas_call(
        _head_body,
        grid=grid,
        in_specs=[
            pl.BlockSpec((BT, MF_DIM), lambda i: (i, 0)),
            pl.BlockSpec((BT, MF_DIM), lambda i: (i, 0)),
            pl.BlockSpec((BT, MLP_IN_HALF), lambda i: (i, 0)),
            pl.BlockSpec((BT, MLP_IN_HALF), lambda i: (i, 0)),
            full((MLP_IN_HALF, H0)),
            full((MLP_IN_HALF, H0)),
            full((1, H0)),
            full((H0, H1)),
            full((1, H1)),
            full((1, MF_DIM)),
            full((1, H1)),
            full((1,)),
        ],
        out_specs=pl.BlockSpec((BT,), lambda i: (i,)),
        out_shape=jax.ShapeDtypeStruct((BATCH,), jnp.float32),
    )(gu, gi, mu, mi, w0u, w0i, b0r, w1, b1r, wfg, wfm, bf)


def kernel(user_idxs, item_idxs, gmf_user, gmf_item, mlp_user, mlp_item,
           W0, b0, W1, b1, Wf, bf):
    gu, gi, mu, mi = _gather_all(gmf_user, gmf_item, mlp_user, mlp_item,
                                 user_idxs, item_idxs)
    return _dense_head(gu, gi, mu, mi, W0, b0, W1, b1, Wf, bf)


# speedup vs baseline: 1.2989x; 1.2989x over previous
import functools

import jax
import jax.numpy as jnp
from jax import lax
from jax.experimental import pallas as pl
from jax.experimental.pallas import tpu as pltpu
from jax.experimental.pallas import tpu_sc as plsc

BATCH = 16384
MF_DIM = 64
MLP_IN_HALF = 128
H0 = 128
H1 = 64

NC, NS = 2, 16
NW = NC * NS
B_PER_W = BATCH // NW
CHUNK = B_PER_W // 2


def _gather_all(gmf_user, gmf_item, mlp_user, mlp_item, user_idxs, item_idxs):
    mesh = plsc.VectorSubcoreMesh(core_axis_name="c", subcore_axis_name="s")

    @functools.partial(
        pl.kernel,
        mesh=mesh,
        out_type=[
            jax.ShapeDtypeStruct((BATCH, MF_DIM), jnp.float32),
            jax.ShapeDtypeStruct((BATCH, MF_DIM), jnp.float32),
            jax.ShapeDtypeStruct((BATCH, MLP_IN_HALF), jnp.float32),
            jax.ShapeDtypeStruct((BATCH, MLP_IN_HALF), jnp.float32),
        ],
        scratch_types=[
            pltpu.VMEM((B_PER_W,), jnp.int32),
            pltpu.VMEM((B_PER_W,), jnp.int32),
            pltpu.VMEM((B_PER_W, MF_DIM), jnp.float32),
            pltpu.VMEM((B_PER_W, MLP_IN_HALF), jnp.float32),
            pltpu.SemaphoreType.DMA,
            pltpu.SemaphoreType.DMA,
            pltpu.SemaphoreType.DMA,
        ],
        compiler_params=pltpu.CompilerParams(use_tc_tiling_on_sc=False),
    )
    def k(gu_hbm, gi_hbm, mu_hbm, mi_hbm, uidx_hbm, iidx_hbm,
          out_gu, out_gi, out_mu, out_mi,
          uidx_v, iidx_v, gbuf, mbuf, sg0, sg1, sm):
        wid = lax.axis_index("s") * NC + lax.axis_index("c")
        base = wid * B_PER_W
        sl = pl.ds(base, B_PER_W)
        pltpu.sync_copy(uidx_hbm.at[sl], uidx_v)
        pltpu.sync_copy(iidx_hbm.at[sl], iidx_v)

        cg0 = pltpu.async_copy(gu_hbm.at[uidx_v], gbuf, sg0)
        cm0 = pltpu.async_copy(mu_hbm.at[uidx_v], mbuf, sm)
        cg0.wait()
        pltpu.sync_copy(gbuf, out_gu.at[sl])
        cg1 = pltpu.async_copy(gi_hbm.at[iidx_v], gbuf, sg1)
        cm0.wait()
        pltpu.sync_copy(mbuf, out_mu.at[sl])
        cm1 = pltpu.async_copy(mi_hbm.at[iidx_v], mbuf, sm)
        cg1.wait()
        pltpu.sync_copy(gbuf, out_gi.at[sl])
        cm1.wait()
        pltpu.sync_copy(mbuf, out_mi.at[sl])

    return k(gmf_user, gmf_item, mlp_user, mlp_item, user_idxs, item_idxs)


BT = 2048


def _head_body(gu_ref, gi_ref, mu_ref, mi_ref,
               w0u_ref, w0i_ref, b0_ref, w1_ref, b1_ref,
               wfg_ref, wfm_ref, bf_ref, out_ref):
    h0 = jnp.dot(mu_ref[...], w0u_ref[...], preferred_element_type=jnp.float32)
    h0 += jnp.dot(mi_ref[...], w0i_ref[...], preferred_element_type=jnp.float32)
    h0 = jnp.maximum(h0 + b0_ref[...], 0.0)
    h1 = jnp.dot(h0, w1_ref[...], preferred_element_type=jnp.float32)
    h1 = jnp.maximum(h1 + b1_ref[...], 0.0)
    gmf = gu_ref[...] * gi_ref[...]
    logit = jnp.sum(gmf * wfg_ref[...], axis=1) + jnp.sum(h1 * wfm_ref[...], axis=1)
    out_ref[...] = jax.nn.sigmoid(logit + bf_ref[0])


def _dense_head(gu, gi, mu, mi, W0, b0, W1, b1, Wf, bf):
    w0u = W0[:, :MLP_IN_HALF].T
    w0i = W0[:, MLP_IN_HALF:].T
    w1 = W1.T
    wfg = Wf[0, :MF_DIM].reshape(1, MF_DIM)
    wfm = Wf[0, MF_DIM:].reshape(1, H1)
    b0r = b0.reshape(1, H0)
    b1r = b1.reshape(1, H1)

    grid = (BATCH // BT,)
    full = lambda shape: pl.BlockSpec(shape, lambda i: (0,) * len(shape))
    return pl.pallas_call(
        _head_body,
        grid=grid,
        in_specs=[
            pl.BlockSpec((BT, MF_DIM), lambda i: (i, 0)),
            pl.BlockSpec((BT, MF_DIM), lambda i: (i, 0)),
            pl.BlockSpec((BT, MLP_IN_HALF), lambda i: (i, 0)),
            pl.BlockSpec((BT, MLP_IN_HALF), lambda i: (i, 0)),
            full((MLP_IN_HALF, H0)),
            full((MLP_IN_HALF, H0)),
            full((1, H0)),
            full((H0, H1)),
            full((1, H1)),
            full((1, MF_DIM)),
            full((1, H1)),
            full((1,)),
        ],
        out_specs=pl.BlockSpec((BT,), lambda i: (i,)),
        out_shape=jax.ShapeDtypeStruct((BATCH,), jnp.float32),
    )(gu, gi, mu, mi, w0u, w0i, b0r, w1, b1r, wfg, wfm, bf)


def kernel(user_idxs, item_idxs, gmf_user, gmf_item, mlp_user, mlp_item,
           W0, b0, W1, b1, Wf, bf):
    gu, gi, mu, mi = _gather_all(gmf_user, gmf_item, mlp_user, mlp_item,
                                 user_idxs, item_idxs)
    return _dense_head(gu, gi, mu, mi, W0, b0, W1, b1, Wf, bf)
